# baseline (speedup 1.0000x reference)
import functools

import jax
import jax.numpy as jnp
from jax import lax
from jax.experimental import pallas as pl
from jax.experimental.pallas import tpu as pltpu
from jax.experimental.pallas import tpu_sc as plsc

BATCH = 16384
NF = 32
NW = 32
B_PER_W = BATCH // NW
CHUNK = 128
N_CHUNKS = B_PER_W // CHUNK
L = 16
N_USERS = 1000000
N_ITEMS = 100000


def _sc_gather_body(uid_hbm, iid_hbm, gu_hbm, gi_hbm, mu_hbm, mi_hbm,
                    gmf_out, mu_out, mi_out,
                    ids_u, ids_i, idx_u, idx_i,
                    e_gu, e_gi, e_u, e_i, sem):
    wid = lax.axis_index("s") * 2 + lax.axis_index("c")
    base = wid * B_PER_W

    pltpu.sync_copy(uid_hbm.at[pl.ds(base, B_PER_W)], ids_u)
    pltpu.sync_copy(iid_hbm.at[pl.ds(base, B_PER_W)], ids_i)

    def chunk_step(k, carry):
        c0 = k * CHUNK

        def build_idx(g, carry2):
            u16 = ids_u[pl.ds(c0 + g * L, L)]
            i16 = ids_i[pl.ds(c0 + g * L, L)]
            col = pl.ds(g * L, L)
            for c in range(NF):
                idx_u[c, 0, col] = u16 + jnp.int32(c * N_USERS)
                idx_i[c, 0, col] = i16 + jnp.int32(c * N_ITEMS)
            return carry2
        lax.fori_loop(0, CHUNK // L, build_idx, 0)

        for c in range(NF):
            pltpu.async_copy(gu_hbm.at[idx_u.at[c, 0]], e_gu.at[c, 0], sem)
            pltpu.async_copy(gi_hbm.at[idx_i.at[c, 0]], e_gi.at[c, 0], sem)
            pltpu.async_copy(mu_hbm.at[idx_u.at[c, 0]], e_u.at[c, 0], sem)
            pltpu.async_copy(mi_hbm.at[idx_i.at[c, 0]], e_i.at[c, 0], sem)

        B = (base + c0) // CHUNK
        dst = pl.ds(B, 1)
        pltpu.make_async_copy(gmf_out.at[:, dst], e_gu, sem).wait()
        pltpu.make_async_copy(gmf_out.at[:, dst], e_gi, sem).wait()
        pltpu.make_async_copy(mu_out.at[:, dst], e_u, sem).wait()
        pltpu.make_async_copy(mi_out.at[:, dst], e_i, sem).wait()

        def mul_step(m, carry2):
            row = m // (CHUNK // L)
            col = pl.ds((m % (CHUNK // L)) * L, L)
            e_gu[row, 0, col] = e_gu[row, 0, col] * e_gi[row, 0, col]
            return carry2
        lax.fori_loop(0, NF * CHUNK // L, mul_step, 0)

        pltpu.sync_copy(e_gu, gmf_out.at[:, dst])
        pltpu.sync_copy(e_u, mu_out.at[:, dst])
        pltpu.sync_copy(e_i, mi_out.at[:, dst])
        return carry
    lax.fori_loop(0, N_CHUNKS, chunk_step, 0)


def _sc_gather(user_ids, item_ids, guF, giF, muF, miF):
    mesh = plsc.VectorSubcoreMesh(core_axis_name="c", subcore_axis_name="s")
    f32 = jnp.float32
    i32 = jnp.int32
    nb = BATCH // CHUNK
    out_type = (
        jax.ShapeDtypeStruct((NF, nb, CHUNK), f32),
        jax.ShapeDtypeStruct((NF, nb, CHUNK), f32),
        jax.ShapeDtypeStruct((NF, nb, CHUNK), f32),
    )
    scratch = [
        pltpu.VMEM((B_PER_W,), i32),
        pltpu.VMEM((B_PER_W,), i32),
        pltpu.VMEM((NF, 1, CHUNK), i32),
        pltpu.VMEM((NF, 1, CHUNK), i32),
        pltpu.VMEM((NF, 1, CHUNK), f32),
        pltpu.VMEM((NF, 1, CHUNK), f32),
        pltpu.VMEM((NF, 1, CHUNK), f32),
        pltpu.VMEM((NF, 1, CHUNK), f32),
        pltpu.SemaphoreType.DMA,
    ]
    fn = pl.kernel(_sc_gather_body, out_type=out_type, mesh=mesh,
                   scratch_types=scratch,
                   compiler_params=pltpu.CompilerParams(
                       use_tc_tiling_on_sc=True))
    return fn(user_ids, item_ids, guF, giF, muF, miF)


BB = 1024
NB = BB // CHUNK


def _tc_mlp_body(gmf_ref, mu_ref, mi_ref, W1_ref, b1_ref, W2_ref, b2_ref,
                 W3_ref, b3_ref, Wo_ref, bo_ref, out_ref):
    f32 = jnp.float32
    cT = (((0,), (0,)), ((), ()))
    w1 = W1_ref[...]
    wo = Wo_ref[...]
    for t in range(NB):
        mu = mu_ref[:, t, :]
        mi = mi_ref[:, t, :]
        h = lax.dot_general(w1[:NF], mu, cT, preferred_element_type=f32)
        h += lax.dot_general(w1[NF:], mi, cT, preferred_element_type=f32)
        h = jnp.maximum(h + b1_ref[...], 0.0)
        h = jnp.maximum(lax.dot_general(W2_ref[...], h, cT,
                                        preferred_element_type=f32)
                        + b2_ref[...], 0.0)
        h = jnp.maximum(lax.dot_general(W3_ref[...], h, cT,
                                        preferred_element_type=f32)
                        + b3_ref[...], 0.0)
        logits = lax.dot_general(gmf_ref[:, t, :], wo[:NF], cT,
                                 preferred_element_type=f32)
        logits += lax.dot_general(h, wo[NF:], cT, preferred_element_type=f32)
        logits += bo_ref[...]
        out_ref[pl.ds(t * CHUNK, CHUNK), :] = jax.nn.sigmoid(logits)


def _tc_mlp(gmf_t, mu_t, mi_t, W1, b1, W2, b2, W3, b3, Wo, bo):
    grid = (BATCH // BB,)
    col_spec = pl.BlockSpec((NF, NB, CHUNK), lambda i: (0, i, 0))
    full = lambda a: pl.BlockSpec(a.shape, lambda i: (0,) * a.ndim)
    return pl.pallas_call(
        _tc_mlp_body,
        grid=grid,
        in_specs=[col_spec, col_spec, col_spec,
                  full(W1), full(b1), full(W2), full(b2),
                  full(W3), full(b3), full(Wo), full(bo)],
        out_specs=pl.BlockSpec((BB, 1), lambda i: (i, 0)),
        out_shape=jax.ShapeDtypeStruct((BATCH, 1), jnp.float32),
        compiler_params=pltpu.CompilerParams(
            dimension_semantics=("arbitrary",)),
    )(gmf_t, mu_t, mi_t, W1, b1, W2, b2, W3, b3, Wo, bo)


def kernel(user_ids, item_ids, gmf_user_emb, gmf_item_emb, mlp_user_emb,
           mlp_item_emb, W1, b1, W2, b2, W3, b3, Wo, bo):
    guF = gmf_user_emb.T.reshape(-1)
    giF = gmf_item_emb.T.reshape(-1)
    muF = mlp_user_emb.T.reshape(-1)
    miF = mlp_item_emb.T.reshape(-1)
    gmf_t, mu_t, mi_t = _sc_gather(user_ids, item_ids, guF, giF, muF, miF)
    b1c = b1.reshape(-1, 1)
    b2c = b2.reshape(-1, 1)
    b3c = b3.reshape(-1, 1)
    return _tc_mlp(gmf_t, mu_t, mi_t, W1, b1c, W2, b2c, W3, b3c,
                   Wo, bo.reshape(-1, 1))


# speedup vs baseline: 1.7375x; 1.7375x over previous
import functools

import jax
import jax.numpy as jnp
from jax import lax
from jax.experimental import pallas as pl
from jax.experimental.pallas import tpu as pltpu
from jax.experimental.pallas import tpu_sc as plsc

BATCH = 16384
NF = 32
NW = 32
B_PER_W = BATCH // NW
CHUNK = 128
N_CHUNKS = B_PER_W // CHUNK
L = 16
N_USERS = 1000000
N_ITEMS = 100000
NBU = 7813
NBI = 782
SU = NBU * 128
SI = NBI * 128
RB = 16384


def _relayout_body(a_ref, b_ref, pa_ref, pb_ref):
    for B in range(RB // 128):
        s = pl.ds(B * 128, 128)
        pa_ref[:, B, :] = a_ref[:, s]
        pb_ref[:, B, :] = b_ref[:, s]


def _tc_relayout(aT, bT, nb):
    n = aT.shape[1]
    grid = ((n + RB - 1) // RB,)
    in_spec = pl.BlockSpec((NF, RB), lambda i: (0, i))
    out_spec = pl.BlockSpec((NF, RB // 128, 128), lambda i: (0, i, 0))
    f32 = jnp.float32
    return pl.pallas_call(
        _relayout_body,
        grid=grid,
        in_specs=[in_spec, in_spec],
        out_specs=[out_spec, out_spec],
        out_shape=[jax.ShapeDtypeStruct((NF, nb, 128), f32)] * 2,
        compiler_params=pltpu.CompilerParams(
            dimension_semantics=("arbitrary",)),
    )(aT, bT)


def _sc_gather_body(uid_hbm, iid_hbm, gu_hbm, gi_hbm, mu_hbm, mi_hbm,
                    gmf_out, mu_out, mi_out,
                    ids_u, ids_i, idx_u, idx_i,
                    e_gu, e_gi, e_u, e_i, sem):
    wid = lax.axis_index("s") * 2 + lax.axis_index("c")
    base = wid * B_PER_W

    pltpu.sync_copy(uid_hbm.at[pl.ds(base, B_PER_W)], ids_u)
    pltpu.sync_copy(iid_hbm.at[pl.ds(base, B_PER_W)], ids_i)

    def chunk_step(k, carry):
        c0 = k * CHUNK

        def build_idx(g, carry2):
            u16 = ids_u[pl.ds(c0 + g * L, L)]
            i16 = ids_i[pl.ds(c0 + g * L, L)]
            col = pl.ds(g * L, L)
            for c in range(NF):
                idx_u[c, 0, col] = u16 + jnp.int32(c * SU)
                idx_i[c, 0, col] = i16 + jnp.int32(c * SI)
            return carry2
        lax.fori_loop(0, CHUNK // L, build_idx, 0)

        for c in range(NF):
            pltpu.async_copy(gu_hbm.at[idx_u.at[c, 0]], e_gu.at[c, 0], sem)
            pltpu.async_copy(gi_hbm.at[idx_i.at[c, 0]], e_gi.at[c, 0], sem)
            pltpu.async_copy(mu_hbm.at[idx_u.at[c, 0]], e_u.at[c, 0], sem)
            pltpu.async_copy(mi_hbm.at[idx_i.at[c, 0]], e_i.at[c, 0], sem)

        B = (base + c0) // CHUNK
        dst = pl.ds(B, 1)
        pltpu.make_async_copy(gmf_out.at[:, dst], e_gu, sem).wait()
        pltpu.make_async_copy(gmf_out.at[:, dst], e_gi, sem).wait()
        pltpu.make_async_copy(mu_out.at[:, dst], e_u, sem).wait()
        pltpu.make_async_copy(mi_out.at[:, dst], e_i, sem).wait()

        def mul_step(m, carry2):
            row = m // (CHUNK // L)
            col = pl.ds((m % (CHUNK // L)) * L, L)
            e_gu[row, 0, col] = e_gu[row, 0, col] * e_gi[row, 0, col]
            return carry2
        lax.fori_loop(0, NF * CHUNK // L, mul_step, 0)

        pltpu.sync_copy(e_gu, gmf_out.at[:, dst])
        pltpu.sync_copy(e_u, mu_out.at[:, dst])
        pltpu.sync_copy(e_i, mi_out.at[:, dst])
        return carry
    lax.fori_loop(0, N_CHUNKS, chunk_step, 0)


def _sc_gather(user_ids, item_ids, guF, giF, muF, miF):
    mesh = plsc.VectorSubcoreMesh(core_axis_name="c", subcore_axis_name="s")
    f32 = jnp.float32
    i32 = jnp.int32
    nb = BATCH // CHUNK
    out_type = (
        jax.ShapeDtypeStruct((NF, nb, CHUNK), f32),
        jax.ShapeDtypeStruct((NF, nb, CHUNK), f32),
        jax.ShapeDtypeStruct((NF, nb, CHUNK), f32),
    )
    scratch = [
        pltpu.VMEM((B_PER_W,), i32),
        pltpu.VMEM((B_PER_W,), i32),
        pltpu.VMEM((NF, 1, CHUNK), i32),
        pltpu.VMEM((NF, 1, CHUNK), i32),
        pltpu.VMEM((NF, 1, CHUNK), f32),
        pltpu.VMEM((NF, 1, CHUNK), f32),
        pltpu.VMEM((NF, 1, CHUNK), f32),
        pltpu.VMEM((NF, 1, CHUNK), f32),
        pltpu.SemaphoreType.DMA,
    ]
    fn = pl.kernel(_sc_gather_body, out_type=out_type, mesh=mesh,
                   scratch_types=scratch,
                   compiler_params=pltpu.CompilerParams(
                       use_tc_tiling_on_sc=True))
    return fn(user_ids, item_ids, guF, giF, muF, miF)


BB = 1024
NB = BB // CHUNK


def _tc_mlp_body(gmf_ref, mu_ref, mi_ref, W1_ref, b1_ref, W2_ref, b2_ref,
                 W3_ref, b3_ref, Wo_ref, bo_ref, out_ref):
    f32 = jnp.float32
    cT = (((0,), (0,)), ((), ()))
    w1 = W1_ref[...]
    wo = Wo_ref[...]
    for t in range(NB):
        mu = mu_ref[:, t, :]
        mi = mi_ref[:, t, :]
        h = lax.dot_general(w1[:NF], mu, cT, preferred_element_type=f32)
        h += lax.dot_general(w1[NF:], mi, cT, preferred_element_type=f32)
        h = jnp.maximum(h + b1_ref[...], 0.0)
        h = jnp.maximum(lax.dot_general(W2_ref[...], h, cT,
                                        preferred_element_type=f32)
                        + b2_ref[...], 0.0)
        h = jnp.maximum(lax.dot_general(W3_ref[...], h, cT,
                                        preferred_element_type=f32)
                        + b3_ref[...], 0.0)
        logits = lax.dot_general(gmf_ref[:, t, :], wo[:NF], cT,
                                 preferred_element_type=f32)
        logits += lax.dot_general(h, wo[NF:], cT, preferred_element_type=f32)
        logits += bo_ref[...]
        out_ref[pl.ds(t * CHUNK, CHUNK), :] = jax.nn.sigmoid(logits)


def _tc_mlp(gmf_t, mu_t, mi_t, W1, b1, W2, b2, W3, b3, Wo, bo):
    grid = (BATCH // BB,)
    col_spec = pl.BlockSpec((NF, NB, CHUNK), lambda i: (0, i, 0))
    full = lambda a: pl.BlockSpec(a.shape, lambda i: (0,) * a.ndim)
    return pl.pallas_call(
        _tc_mlp_body,
        grid=grid,
        in_specs=[col_spec, col_spec, col_spec,
                  full(W1), full(b1), full(W2), full(b2),
                  full(W3), full(b3), full(Wo), full(bo)],
        out_specs=pl.BlockSpec((BB, 1), lambda i: (i, 0)),
        out_shape=jax.ShapeDtypeStruct((BATCH, 1), jnp.float32),
        compiler_params=pltpu.CompilerParams(
            dimension_semantics=("arbitrary",)),
    )(gmf_t, mu_t, mi_t, W1, b1, W2, b2, W3, b3, Wo, bo)


def kernel(user_ids, item_ids, gmf_user_emb, gmf_item_emb, mlp_user_emb,
           mlp_item_emb, W1, b1, W2, b2, W3, b3, Wo, bo):
    pgu, pmu = _tc_relayout(gmf_user_emb.T, mlp_user_emb.T, NBU)
    pgi, pmi = _tc_relayout(gmf_item_emb.T, mlp_item_emb.T, NBI)
    guF = pgu.reshape(-1)
    giF = pgi.reshape(-1)
    muF = pmu.reshape(-1)
    miF = pmi.reshape(-1)
    gmf_t, mu_t, mi_t = _sc_gather(user_ids, item_ids, guF, giF, muF, miF)
    b1c = b1.reshape(-1, 1)
    b2c = b2.reshape(-1, 1)
    b3c = b3.reshape(-1, 1)
    return _tc_mlp(gmf_t, mu_t, mi_t, W1, b1c, W2, b2c, W3, b3c,
                   Wo, bo.reshape(-1, 1))


# speedup vs baseline: 5.2024x; 2.9943x over previous
import functools

import jax
import jax.numpy as jnp
from jax import lax
from jax.experimental import pallas as pl
from jax.experimental.pallas import tpu as pltpu
from jax.experimental.pallas import tpu_sc as plsc

BATCH = 16384
NF = 32
NW = 32
B_PER_W = BATCH // NW
CHUNK = 128
N_CHUNKS = B_PER_W // CHUNK
L = 16


def _sc_gather_body(uid_hbm, iid_hbm, gu_hbm, gi_hbm, mu_hbm, mi_hbm,
                    gmf_out, mu_out, mi_out,
                    ids_u_s, ids_i_s,
                    a_gu, a_gi, a_mu, a_mi, e_gu, e_u, e_i, sem):
    wid = lax.axis_index("s") * 2 + lax.axis_index("c")
    base = wid * B_PER_W

    pltpu.sync_copy(uid_hbm.at[pl.ds(base, B_PER_W)], ids_u_s)
    pltpu.sync_copy(iid_hbm.at[pl.ds(base, B_PER_W)], ids_i_s)

    def chunk_step(k, carry):
        c0 = k * CHUNK

        def issue(g, carry2):
            u_vec = ids_u_s[pl.ds(c0 + g * L, L)]
            i_vec = ids_i_s[pl.ds(c0 + g * L, L)]
            for l in range(L):
                j = g * L + l
                u = lax.shift_right_logical(u_vec[l], 2)
                i = lax.shift_right_logical(i_vec[l], 2)
                pltpu.make_async_copy(gu_hbm.at[u], a_gu.at[j], sem).start()
                pltpu.make_async_copy(gi_hbm.at[i], a_gi.at[j], sem).start()
                pltpu.make_async_copy(mu_hbm.at[u], a_mu.at[j], sem).start()
                pltpu.make_async_copy(mi_hbm.at[i], a_mi.at[j], sem).start()
            return carry2
        lax.fori_loop(0, CHUNK // L, issue, 0)

        rows = pl.ds(base + c0, CHUNK)
        pltpu.make_async_copy(gu_hbm.at[pl.ds(0, CHUNK)], a_gu, sem).wait()
        pltpu.make_async_copy(gu_hbm.at[pl.ds(0, CHUNK)], a_gi, sem).wait()
        pltpu.make_async_copy(gu_hbm.at[pl.ds(0, CHUNK)], a_mu, sem).wait()
        pltpu.make_async_copy(gu_hbm.at[pl.ds(0, CHUNK)], a_mi, sem).wait()

        def extract(g, carry2):
            u_vec = ids_u_s[pl.ds(c0 + g * L, L)]
            i_vec = ids_i_s[pl.ds(c0 + g * L, L)]
            for l in range(L):
                j = g * L + l
                qu = lax.bitwise_and(u_vec[l], 3) * NF
                qi = lax.bitwise_and(i_vec[l], 3) * NF
                for h in range(NF // L):
                    d = pl.ds(h * L, L)
                    su = pl.ds(qu + h * L, L)
                    si = pl.ds(qi + h * L, L)
                    e_gu[j, d] = a_gu[j, su] * a_gi[j, si]
                    e_u[j, d] = a_mu[j, su]
                    e_i[j, d] = a_mi[j, si]
            return carry2
        lax.fori_loop(0, CHUNK // L, extract, 0)

        pltpu.sync_copy(e_gu, gmf_out.at[rows])
        pltpu.sync_copy(e_u, mu_out.at[rows])
        pltpu.sync_copy(e_i, mi_out.at[rows])
        return carry
    lax.fori_loop(0, N_CHUNKS, chunk_step, 0)


def _sc_gather(user_ids, item_ids, gu3, gi3, mu3, mi3):
    mesh = plsc.VectorSubcoreMesh(core_axis_name="c", subcore_axis_name="s")
    f32 = jnp.float32
    i32 = jnp.int32
    out_type = (
        jax.ShapeDtypeStruct((BATCH, NF), f32),
        jax.ShapeDtypeStruct((BATCH, NF), f32),
        jax.ShapeDtypeStruct((BATCH, NF), f32),
    )
    scratch = [
        pltpu.VMEM((B_PER_W,), i32),
        pltpu.VMEM((B_PER_W,), i32),
        pltpu.VMEM((CHUNK, 128), f32),
        pltpu.VMEM((CHUNK, 128), f32),
        pltpu.VMEM((CHUNK, 128), f32),
        pltpu.VMEM((CHUNK, 128), f32),
        pltpu.VMEM((CHUNK, NF), f32),
        pltpu.VMEM((CHUNK, NF), f32),
        pltpu.VMEM((CHUNK, NF), f32),
        pltpu.SemaphoreType.DMA,
    ]
    fn = pl.kernel(_sc_gather_body, out_type=out_type, mesh=mesh,
                   scratch_types=scratch,
                   compiler_params=pltpu.CompilerParams(
                       use_tc_tiling_on_sc=True))
    return fn(user_ids, item_ids, gu3, gi3, mu3, mi3)


BB = 1024


def _tc_mlp_body(gmf_ref, mu_ref, mi_ref, W1_ref, b1_ref, W2_ref, b2_ref,
                 W3_ref, b3_ref, Wo_ref, bo_ref, out_ref):
    f32 = jnp.float32
    w1 = W1_ref[...]
    h = jnp.dot(mu_ref[...], w1[:NF], preferred_element_type=f32)
    h += jnp.dot(mi_ref[...], w1[NF:], preferred_element_type=f32)
    h = jnp.maximum(h + b1_ref[...], 0.0)
    h = jnp.maximum(jnp.dot(h, W2_ref[...], preferred_element_type=f32)
                    + b2_ref[...], 0.0)
    h = jnp.maximum(jnp.dot(h, W3_ref[...], preferred_element_type=f32)
                    + b3_ref[...], 0.0)
    wo = Wo_ref[...]
    logits = jnp.dot(gmf_ref[...], wo[:NF], preferred_element_type=f32)
    logits += jnp.dot(h, wo[NF:], preferred_element_type=f32)
    logits += bo_ref[...]
    out_ref[...] = jax.nn.sigmoid(logits)


def _tc_mlp(gmf_vec, mu_rows, mi_rows, W1, b1, W2, b2, W3, b3, Wo, bo):
    grid = (BATCH // BB,)
    row_spec = pl.BlockSpec((BB, NF), lambda i: (i, 0))
    full = lambda a: pl.BlockSpec(a.shape, lambda i: (0,) * a.ndim)
    return pl.pallas_call(
        _tc_mlp_body,
        grid=grid,
        in_specs=[row_spec, row_spec, row_spec,
                  full(W1), full(b1), full(W2), full(b2),
                  full(W3), full(b3), full(Wo), full(bo)],
        out_specs=pl.BlockSpec((BB, 1), lambda i: (i, 0)),
        out_shape=jax.ShapeDtypeStruct((BATCH, 1), jnp.float32),
        compiler_params=pltpu.CompilerParams(
            dimension_semantics=("arbitrary",)),
    )(gmf_vec, mu_rows, mi_rows, W1, b1, W2, b2, W3, b3, Wo, bo)


def kernel(user_ids, item_ids, gmf_user_emb, gmf_item_emb, mlp_user_emb,
           mlp_item_emb, W1, b1, W2, b2, W3, b3, Wo, bo):
    gu3 = gmf_user_emb.reshape(-1, 128)
    gi3 = gmf_item_emb.reshape(-1, 128)
    mu3 = mlp_user_emb.reshape(-1, 128)
    mi3 = mlp_item_emb.reshape(-1, 128)
    gmf_vec, mu_rows, mi_rows = _sc_gather(
        user_ids, item_ids, gu3, gi3, mu3, mi3)
    b1r = b1.reshape(1, -1)
    b2r = b2.reshape(1, -1)
    b3r = b3.reshape(1, -1)
    return _tc_mlp(gmf_vec, mu_rows, mi_rows, W1, b1r, W2, b2r, W3, b3r,
                   Wo, bor := bo.reshape(1, -1))
